# baseline (speedup 1.0000x reference)
import jax
import jax.numpy as jnp
from jax.experimental import pallas as pl


def _add_kernel(scale, x_ref, t_ref, o_ref):
    o_ref[...] = x_ref[...] + t_ref[...] * scale


def kernel(inputs, lookup_table):
    B, T, D = inputs.shape
    F = T * D
    scale = float(D) ** 0.5
    x = inputs.reshape(B, F)
    table = lookup_table.reshape(1, F)
    Bb = 256
    out = pl.pallas_call(
        lambda x_ref, t_ref, o_ref: _add_kernel(scale, x_ref, t_ref, o_ref),
        grid=(B // Bb,),
        in_specs=[
            pl.BlockSpec((Bb, F), lambda i: (i, 0)),
            pl.BlockSpec((1, F), lambda i: (0, 0)),
        ],
        out_specs=pl.BlockSpec((Bb, F), lambda i: (i, 0)),
        out_shape=jax.ShapeDtypeStruct((B, F), jnp.float32),
    )(x, table)
    return out.reshape(B, T, D)
